# Initial kernel scaffold; baseline (speedup 1.0000x reference)
import jax
import jax.numpy as jnp
from jax.experimental import pallas as pl


def kernel(inputs, table, W, b):
    raise NotImplementedError("write your pallas kernel here")


# speedup vs baseline: 1.9609x; 1.9609x over previous
import functools

import jax
import jax.numpy as jnp
from jax import lax
from jax.experimental import pallas as pl
from jax.experimental.pallas import tpu as pltpu
from jax.experimental.pallas import tpu_sc as plsc


def _sc_embed_mean(inputs, table):
    B, L = inputs.shape
    V, D = table.shape
    assert L % 2 == 0 and D % 16 == 0
    half = L // 2
    assert half <= 128
    inputs2 = inputs.reshape(B * 2, half)

    NC, NS = 2, 16
    NW = NC * NS
    assert B % NW == 0
    b_per_w = B // NW
    CB = 64
    assert b_per_w % CB == 0
    nchunks = b_per_w // CB
    ncol = D // 16
    scale = 1.0 / L

    mesh = plsc.VectorSubcoreMesh(core_axis_name="c", subcore_axis_name="s")

    @functools.partial(
        pl.kernel,
        mesh=mesh,
        compiler_params=pltpu.CompilerParams(use_tc_tiling_on_sc=False),
        out_type=jax.ShapeDtypeStruct((B, D), jnp.float32),
        scratch_types=[
            pltpu.VMEM((2 * CB, half), jnp.int32),
            pltpu.VMEM((L, D), jnp.float32),
            pltpu.VMEM((CB, D), jnp.float32),
            pltpu.SemaphoreType.DMA,
        ],
    )
    def k(inputs_hbm, table_hbm, out_hbm, idx_v, rows_v, m_v, sem):
        wid = lax.axis_index("s") * NC + lax.axis_index("c")
        base = wid * b_per_w

        def chunk_body(oc, carry):
            elem0 = base + oc * CB
            pltpu.sync_copy(inputs_hbm.at[pl.ds(elem0 * 2, 2 * CB)], idx_v)

            def elem_body(e, carry):
                g0 = pltpu.async_copy(
                    table_hbm.at[idx_v.at[2 * e]],
                    rows_v.at[pl.ds(0, half)], sem)
                g1 = pltpu.async_copy(
                    table_hbm.at[idx_v.at[2 * e + 1]],
                    rows_v.at[pl.ds(half, half)], sem)
                g0.wait()
                g1.wait()

                def acc_body(j, acc):
                    return tuple(
                        acc[c] + rows_v[j, pl.ds(16 * c, 16)]
                        for c in range(ncol))

                zero = jnp.zeros((16,), jnp.float32)
                acc = lax.fori_loop(0, L, acc_body, (zero,) * ncol)
                for c in range(ncol):
                    m_v[e, pl.ds(16 * c, 16)] = acc[c] * scale
                return carry

            lax.fori_loop(0, CB, elem_body, 0)
            pltpu.sync_copy(m_v, out_hbm.at[pl.ds(elem0, CB)])
            return carry

        lax.fori_loop(0, nchunks, chunk_body, 0)

    return k(inputs2, table)


def _tc_head(m, W, b):
    B, D = m.shape
    LABELS = W.shape[0]
    LP = 1024
    Wp = jnp.zeros((LP, D), jnp.float32).at[:LABELS].set(W)
    bp = jnp.full((1, LP), -1e30, jnp.float32).at[0, :LABELS].set(b)
    BM = 2048

    def body(m_ref, w_ref, b_ref, o_ref):
        logits = lax.dot_general(
            m_ref[...], w_ref[...], (((1,), (1,)), ((), ())),
            preferred_element_type=jnp.float32)
        logits = logits + b_ref[...]
        mx = jnp.max(logits, axis=-1, keepdims=True)
        e = jnp.exp(logits - mx)
        o_ref[...] = e / jnp.sum(e, axis=-1, keepdims=True)

    out = pl.pallas_call(
        body,
        grid=(B // BM,),
        in_specs=[
            pl.BlockSpec((BM, D), lambda i: (i, 0)),
            pl.BlockSpec((LP, D), lambda i: (0, 0)),
            pl.BlockSpec((1, LP), lambda i: (0, 0)),
        ],
        out_specs=pl.BlockSpec((BM, LP), lambda i: (i, 0)),
        out_shape=jax.ShapeDtypeStruct((B, LP), jnp.float32),
    )(m, Wp, bp)
    return out[:, :LABELS]


def kernel(inputs, table, W, b):
    inputs = inputs.astype(jnp.int32)
    m = _sc_embed_mean(inputs, table)
    return _tc_head(m, W, b)


# speedup vs baseline: 2.4537x; 1.2513x over previous
import functools

import jax
import jax.numpy as jnp
from jax import lax
from jax.experimental import pallas as pl
from jax.experimental.pallas import tpu as pltpu
from jax.experimental.pallas import tpu_sc as plsc


def _sc_embed_mean(inputs, table):
    B, L = inputs.shape
    V, D = table.shape
    assert L % 2 == 0 and D % 16 == 0
    half = L // 2
    assert half <= 128
    inputs2 = inputs.reshape(B * 2, half)

    NC, NS = 2, 16
    NW = NC * NS
    assert B % NW == 0
    b_per_w = B // NW
    CB = 64
    assert b_per_w % CB == 0
    nchunks = b_per_w // CB
    ncol = D // 16
    scale = 1.0 / L
    UN = 8
    assert L % UN == 0

    mesh = plsc.VectorSubcoreMesh(core_axis_name="c", subcore_axis_name="s")

    @functools.partial(
        pl.kernel,
        mesh=mesh,
        compiler_params=pltpu.CompilerParams(use_tc_tiling_on_sc=False),
        out_type=jax.ShapeDtypeStruct((B, D), jnp.float32),
        scratch_types=[
            pltpu.VMEM((2 * CB, half), jnp.int32),
            pltpu.VMEM((L, D), jnp.float32),
            pltpu.VMEM((L, D), jnp.float32),
            pltpu.VMEM((CB, D), jnp.float32),
            pltpu.SemaphoreType.DMA,
        ],
    )
    def k(inputs_hbm, table_hbm, out_hbm, idx_v, rows_a, rows_b, m_v, sem):
        wid = lax.axis_index("s") * NC + lax.axis_index("c")
        base = wid * b_per_w

        def fire(e, buf):
            pltpu.async_copy(
                table_hbm.at[idx_v.at[2 * e]], buf.at[pl.ds(0, half)], sem)
            pltpu.async_copy(
                table_hbm.at[idx_v.at[2 * e + 1]],
                buf.at[pl.ds(half, half)], sem)

        def drain(buf):
            pltpu.make_async_copy(
                table_hbm.at[idx_v.at[0]], buf.at[pl.ds(0, half)], sem).wait()
            pltpu.make_async_copy(
                table_hbm.at[idx_v.at[0]],
                buf.at[pl.ds(half, half)], sem).wait()

        def accum(e, buf):
            def grp(g, acc):
                j0 = g * UN
                out = []
                for c in range(ncol):
                    r = [buf[j0 + u, pl.ds(16 * c, 16)] for u in range(UN)]
                    t01 = r[0] + r[1]
                    t23 = r[2] + r[3]
                    t45 = r[4] + r[5]
                    t67 = r[6] + r[7]
                    out.append(acc[c] + ((t01 + t23) + (t45 + t67)))
                return tuple(out)

            zero = jnp.zeros((16,), jnp.float32)
            acc = lax.fori_loop(0, L // UN, grp, (zero,) * ncol)
            for c in range(ncol):
                m_v[e, pl.ds(16 * c, 16)] = acc[c] * scale

        def chunk_body(oc, carry):
            elem0 = base + oc * CB
            pltpu.sync_copy(inputs_hbm.at[pl.ds(elem0 * 2, 2 * CB)], idx_v)
            fire(0, rows_a)

            def pair_body(p, carry):
                e0 = 2 * p
                drain(rows_a)
                fire(e0 + 1, rows_b)
                accum(e0, rows_a)
                drain(rows_b)

                @pl.when(p < CB // 2 - 1)
                def _():
                    fire(e0 + 2, rows_a)

                accum(e0 + 1, rows_b)
                return carry

            lax.fori_loop(0, CB // 2, pair_body, 0)
            pltpu.sync_copy(m_v, out_hbm.at[pl.ds(elem0, CB)])
            return carry

        lax.fori_loop(0, nchunks, chunk_body, 0)

    return k(inputs2, table)


def _tc_head(m, W, b):
    B, D = m.shape
    LABELS = W.shape[0]
    LP = 1024
    Wp = jnp.zeros((LP, D), jnp.float32).at[:LABELS].set(W)
    bp = jnp.full((1, LP), -1e30, jnp.float32).at[0, :LABELS].set(b)
    BM = 2048

    def body(m_ref, w_ref, b_ref, o_ref):
        logits = lax.dot_general(
            m_ref[...], w_ref[...], (((1,), (1,)), ((), ())),
            preferred_element_type=jnp.float32)
        logits = logits + b_ref[...]
        mx = jnp.max(logits, axis=-1, keepdims=True)
        e = jnp.exp(logits - mx)
        o_ref[...] = e / jnp.sum(e, axis=-1, keepdims=True)

    out = pl.pallas_call(
        body,
        grid=(B // BM,),
        in_specs=[
            pl.BlockSpec((BM, D), lambda i: (i, 0)),
            pl.BlockSpec((LP, D), lambda i: (0, 0)),
            pl.BlockSpec((1, LP), lambda i: (0, 0)),
        ],
        out_specs=pl.BlockSpec((BM, LP), lambda i: (i, 0)),
        out_shape=jax.ShapeDtypeStruct((B, LP), jnp.float32),
    )(m, Wp, bp)
    return out[:, :LABELS]


def kernel(inputs, table, W, b):
    inputs = inputs.astype(jnp.int32)
    m = _sc_embed_mean(inputs, table)
    return _tc_head(m, W, b)


# speedup vs baseline: 2.4597x; 1.0025x over previous
import functools

import jax
import jax.numpy as jnp
from jax import lax
from jax.experimental import pallas as pl
from jax.experimental.pallas import tpu as pltpu
from jax.experimental.pallas import tpu_sc as plsc


def _sc_embed_mean(inputs, table):
    B, L = inputs.shape
    V, D = table.shape
    assert L % 2 == 0 and D % 16 == 0
    ha = (L // 2 + 7) // 8 * 8
    hb = L - ha
    assert ha % 8 == 0 and ha <= 128 and hb <= 128

    NC, NS = 2, 16
    NW = NC * NS
    assert B % NW == 0
    b_per_w = B // NW
    CB = 64
    assert b_per_w % CB == 0
    nchunks = b_per_w // CB
    ncol = D // 16
    scale = 1.0 / L
    UN = 8
    assert L % UN == 0

    mesh = plsc.VectorSubcoreMesh(core_axis_name="c", subcore_axis_name="s")

    @functools.partial(
        pl.kernel,
        mesh=mesh,
        compiler_params=pltpu.CompilerParams(use_tc_tiling_on_sc=False),
        out_type=jax.ShapeDtypeStruct((B, D), jnp.float32),
        scratch_types=[
            pltpu.VMEM((CB, L), jnp.int32),
            pltpu.VMEM((L, D), jnp.float32),
            pltpu.VMEM((L, D), jnp.float32),
            pltpu.VMEM((CB, D), jnp.float32),
            pltpu.SemaphoreType.DMA,
        ],
    )
    def k(inputs_hbm, table_hbm, out_hbm, idx_v, rows_a, rows_b, m_v, sem):
        wid = lax.axis_index("s") * NC + lax.axis_index("c")
        base = wid * b_per_w

        def fire(e, buf):
            pltpu.async_copy(
                table_hbm.at[idx_v.at[e, pl.ds(0, ha)]],
                buf.at[pl.ds(0, ha)], sem)
            pltpu.async_copy(
                table_hbm.at[idx_v.at[e, pl.ds(ha, hb)]],
                buf.at[pl.ds(ha, hb)], sem)

        def drain(buf):
            pltpu.make_async_copy(
                table_hbm.at[idx_v.at[0, pl.ds(0, ha)]],
                buf.at[pl.ds(0, ha)], sem).wait()
            pltpu.make_async_copy(
                table_hbm.at[idx_v.at[0, pl.ds(ha, hb)]],
                buf.at[pl.ds(ha, hb)], sem).wait()

        def accum(e, buf):
            def grp(g, acc):
                j0 = g * UN
                out = []
                for c in range(ncol):
                    r = [buf[j0 + u, pl.ds(16 * c, 16)] for u in range(UN)]
                    t01 = r[0] + r[1]
                    t23 = r[2] + r[3]
                    t45 = r[4] + r[5]
                    t67 = r[6] + r[7]
                    out.append(acc[c] + ((t01 + t23) + (t45 + t67)))
                return tuple(out)

            zero = jnp.zeros((16,), jnp.float32)
            acc = lax.fori_loop(0, L // UN, grp, (zero,) * ncol)
            for c in range(ncol):
                m_v[e, pl.ds(16 * c, 16)] = acc[c] * scale

        def chunk_body(oc, carry):
            elem0 = base + oc * CB
            pltpu.sync_copy(inputs_hbm.at[pl.ds(elem0, CB)], idx_v)
            fire(0, rows_a)

            def pair_body(p, carry):
                e0 = 2 * p
                drain(rows_a)
                fire(e0 + 1, rows_b)
                accum(e0, rows_a)
                drain(rows_b)

                @pl.when(p < CB // 2 - 1)
                def _():
                    fire(e0 + 2, rows_a)

                accum(e0 + 1, rows_b)
                return carry

            lax.fori_loop(0, CB // 2, pair_body, 0)
            pltpu.sync_copy(m_v, out_hbm.at[pl.ds(elem0, CB)])
            return carry

        lax.fori_loop(0, nchunks, chunk_body, 0)

    return k(inputs, table)


def _tc_head(m, W, b):
    B, D = m.shape
    LABELS = W.shape[0]
    LP = 1024
    Wp = jnp.zeros((LP, D), jnp.float32).at[:LABELS].set(W)
    bp = jnp.full((1, LP), -1e30, jnp.float32).at[0, :LABELS].set(b)
    BM = 2048

    def body(m_ref, w_ref, b_ref, o_ref):
        logits = lax.dot_general(
            m_ref[...], w_ref[...], (((1,), (1,)), ((), ())),
            preferred_element_type=jnp.float32)
        logits = logits + b_ref[...]
        mx = jnp.max(logits, axis=-1, keepdims=True)
        e = jnp.exp(logits - mx)
        p = e / jnp.sum(e, axis=-1, keepdims=True)
        o_ref[...] = p[:, :LABELS]

    out = pl.pallas_call(
        body,
        grid=(B // BM,),
        in_specs=[
            pl.BlockSpec((BM, D), lambda i: (i, 0)),
            pl.BlockSpec((LP, D), lambda i: (0, 0)),
            pl.BlockSpec((1, LP), lambda i: (0, 0)),
        ],
        out_specs=pl.BlockSpec((BM, LABELS), lambda i: (i, 0)),
        out_shape=jax.ShapeDtypeStruct((B, LABELS), jnp.float32),
    )(m, Wp, bp)
    return out


def kernel(inputs, table, W, b):
    inputs = inputs.astype(jnp.int32)
    m = _sc_embed_mean(inputs, table)
    return _tc_head(m, W, b)


# speedup vs baseline: 2.5689x; 1.0444x over previous
import functools

import jax
import jax.numpy as jnp
from jax import lax
from jax.experimental import pallas as pl
from jax.experimental.pallas import tpu as pltpu
from jax.experimental.pallas import tpu_sc as plsc


def _sc_embed_mean(inputs, table):
    B, L = inputs.shape
    V, D = table.shape
    assert L % 2 == 0 and D % 16 == 0
    ha = (L // 2 + 7) // 8 * 8
    hb = L - ha
    assert ha % 8 == 0 and ha <= 128 and hb <= 128

    NC, NS = 2, 16
    NW = NC * NS
    assert B % NW == 0
    b_per_w = B // NW
    CB = 64
    assert b_per_w % CB == 0
    nchunks = b_per_w // CB
    ncol = D // 16
    scale = 1.0 / L
    UN = 8
    assert L % UN == 0

    mesh = plsc.VectorSubcoreMesh(core_axis_name="c", subcore_axis_name="s")

    @functools.partial(
        pl.kernel,
        mesh=mesh,
        compiler_params=pltpu.CompilerParams(use_tc_tiling_on_sc=False),
        out_type=jax.ShapeDtypeStruct((B, D), jnp.float32),
        scratch_types=[
            pltpu.VMEM((CB, L), jnp.int32),
            pltpu.VMEM((L, D), jnp.float32),
            pltpu.VMEM((L, D), jnp.float32),
            pltpu.VMEM((CB, D), jnp.float32),
            pltpu.SemaphoreType.DMA,
        ],
    )
    def k(inputs_hbm, table_hbm, out_hbm, idx_v, rows_a, rows_b, m_v, sem):
        wid = lax.axis_index("s") * NC + lax.axis_index("c")
        base = wid * b_per_w

        def fire(e, buf):
            pltpu.async_copy(
                table_hbm.at[idx_v.at[e, pl.ds(0, ha)]],
                buf.at[pl.ds(0, ha)], sem)
            pltpu.async_copy(
                table_hbm.at[idx_v.at[e, pl.ds(ha, hb)]],
                buf.at[pl.ds(ha, hb)], sem)

        def drain(buf):
            pltpu.make_async_copy(
                table_hbm.at[idx_v.at[0, pl.ds(0, ha)]],
                buf.at[pl.ds(0, ha)], sem).wait()
            pltpu.make_async_copy(
                table_hbm.at[idx_v.at[0, pl.ds(ha, hb)]],
                buf.at[pl.ds(ha, hb)], sem).wait()

        def accum(e, buf):
            def grp(g, acc):
                j0 = g * UN
                out = []
                for c in range(ncol):
                    r = [buf[j0 + u, pl.ds(16 * c, 16)] for u in range(UN)]
                    t01 = r[0] + r[1]
                    t23 = r[2] + r[3]
                    t45 = r[4] + r[5]
                    t67 = r[6] + r[7]
                    out.append(acc[c] + ((t01 + t23) + (t45 + t67)))
                return tuple(out)

            zero = jnp.zeros((16,), jnp.float32)
            acc = lax.fori_loop(0, L // UN, grp, (zero,) * ncol)
            for c in range(ncol):
                m_v[e, pl.ds(16 * c, 16)] = acc[c] * scale

        def chunk_body(oc, carry):
            elem0 = base + oc * CB
            pltpu.sync_copy(inputs_hbm.at[pl.ds(elem0, CB)], idx_v)
            fire(0, rows_a)

            def pair_body(p, carry):
                e0 = 2 * p
                drain(rows_a)
                fire(e0 + 1, rows_b)
                accum(e0, rows_a)
                drain(rows_b)

                @pl.when(p < CB // 2 - 1)
                def _():
                    fire(e0 + 2, rows_a)

                accum(e0 + 1, rows_b)
                return carry

            lax.fori_loop(0, CB // 2, pair_body, 0)
            pltpu.sync_copy(m_v, out_hbm.at[pl.ds(elem0, CB)])
            return carry

        lax.fori_loop(0, nchunks, chunk_body, 0)

    return k(inputs, table)


def _tc_head(m, W, b):
    B, D = m.shape
    LABELS = W.shape[0]
    LP = 1024
    Wp = jnp.zeros((LP, D), jnp.float32).at[:LABELS].set(W)
    bp = jnp.full((LP, 1), -1e30, jnp.float32).at[:LABELS, 0].set(b)
    BM = 2048

    def body(m_ref, w_ref, b_ref, o_ref):
        logits = lax.dot_general(
            w_ref[...], m_ref[...], (((1,), (1,)), ((), ())),
            preferred_element_type=jnp.float32)
        logits = logits + b_ref[...]
        mx = jnp.max(logits, axis=0, keepdims=True)
        e = jnp.exp(logits - mx)
        p = e / jnp.sum(e, axis=0, keepdims=True)
        o_ref[...] = p[:LABELS, :]

    out = pl.pallas_call(
        body,
        grid=(B // BM,),
        in_specs=[
            pl.BlockSpec((BM, D), lambda i: (i, 0)),
            pl.BlockSpec((LP, D), lambda i: (0, 0)),
            pl.BlockSpec((LP, 1), lambda i: (0, 0)),
        ],
        out_specs=pl.BlockSpec((LABELS, BM), lambda i: (0, i)),
        out_shape=jax.ShapeDtypeStruct((LABELS, B), jnp.float32),
    )(m, Wp, bp)
    return out.T


def kernel(inputs, table, W, b):
    inputs = inputs.astype(jnp.int32)
    m = _sc_embed_mean(inputs, table)
    return _tc_head(m, W, b)


# speedup vs baseline: 3.4610x; 1.3472x over previous
import functools

import jax
import jax.numpy as jnp
from jax import lax
from jax.experimental import pallas as pl
from jax.experimental.pallas import tpu as pltpu
from jax.experimental.pallas import tpu_sc as plsc


def _sc_embed_mean(inputs, table):
    B, L = inputs.shape
    V, D = table.shape
    assert L % 2 == 0 and D % 16 == 0
    ha = (L // 2 + 7) // 8 * 8
    hb = L - ha
    assert ha % 8 == 0 and ha <= 128 and hb <= 128

    NC, NS = 2, 16
    NW = NC * NS
    assert B % NW == 0
    b_per_w = B // NW
    CB = 128
    assert b_per_w % CB == 0
    nchunks = b_per_w // CB
    ncol = D // 16
    scale = 1.0 / L
    UN = 8
    assert L % UN == 0

    mesh = plsc.VectorSubcoreMesh(core_axis_name="c", subcore_axis_name="s")

    @functools.partial(
        pl.kernel,
        mesh=mesh,
        compiler_params=pltpu.CompilerParams(use_tc_tiling_on_sc=False),
        out_type=jax.ShapeDtypeStruct((B, D), jnp.float32),
        scratch_types=[
            pltpu.VMEM((CB, L), jnp.int32),
            pltpu.VMEM((L, D), jnp.float32),
            pltpu.VMEM((L, D), jnp.float32),
            pltpu.VMEM((L, D), jnp.float32),
            pltpu.VMEM((L, D), jnp.float32),
            pltpu.VMEM((CB, D), jnp.float32),
            pltpu.SemaphoreType.DMA,
        ],
    )
    def k(inputs_hbm, table_hbm, out_hbm, idx_v, r0, r1, r2, r3, m_v, sem):
        wid = lax.axis_index("s") * NC + lax.axis_index("c")
        base = wid * b_per_w
        bufs = (r0, r1, r2, r3)

        def fire(e, buf):
            pltpu.async_copy(
                table_hbm.at[idx_v.at[e, pl.ds(0, ha)]],
                buf.at[pl.ds(0, ha)], sem)
            pltpu.async_copy(
                table_hbm.at[idx_v.at[e, pl.ds(ha, hb)]],
                buf.at[pl.ds(ha, hb)], sem)

        def drain(buf):
            pltpu.make_async_copy(
                table_hbm.at[idx_v.at[0, pl.ds(0, ha)]],
                buf.at[pl.ds(0, ha)], sem).wait()
            pltpu.make_async_copy(
                table_hbm.at[idx_v.at[0, pl.ds(ha, hb)]],
                buf.at[pl.ds(ha, hb)], sem).wait()

        def accum(e, buf):
            def grp(g, acc):
                j0 = g * UN
                out = []
                for c in range(ncol):
                    r = [buf[j0 + u, pl.ds(16 * c, 16)] for u in range(UN)]
                    t01 = r[0] + r[1]
                    t23 = r[2] + r[3]
                    t45 = r[4] + r[5]
                    t67 = r[6] + r[7]
                    out.append(acc[c] + ((t01 + t23) + (t45 + t67)))
                return tuple(out)

            zero = jnp.zeros((16,), jnp.float32)
            acc = lax.fori_loop(0, L // UN, grp, (zero,) * ncol)
            for c in range(ncol):
                m_v[e, pl.ds(16 * c, 16)] = acc[c] * scale

        def chunk_body(oc, carry):
            elem0 = base + oc * CB
            pltpu.sync_copy(inputs_hbm.at[pl.ds(elem0, CB)], idx_v)
            fire(0, r0)
            fire(1, r1)
            fire(2, r2)

            def quad_body(q, carry):
                e0 = 4 * q
                for j in range(4):
                    buf = bufs[j]
                    nxt = bufs[(j + 3) % 4]
                    drain(buf)

                    @pl.when(e0 + j + 3 < CB)
                    def _(e=e0 + j + 3, nb=nxt):
                        fire(e, nb)

                    accum(e0 + j, buf)
                return carry

            lax.fori_loop(0, CB // 4, quad_body, 0)
            pltpu.sync_copy(m_v, out_hbm.at[pl.ds(elem0, CB)])
            return carry

        lax.fori_loop(0, nchunks, chunk_body, 0)

    return k(inputs, table)


def _tc_head(m, W, b):
    B, D = m.shape
    LABELS = W.shape[0]
    LP = 1024
    Wp = jnp.zeros((LP, D), jnp.float32).at[:LABELS].set(W)
    bp = jnp.full((LP, 1), -1e30, jnp.float32).at[:LABELS, 0].set(b)
    BM = 2048

    def body(m_ref, w_ref, b_ref, o_ref):
        logits = lax.dot_general(
            w_ref[...], m_ref[...], (((1,), (1,)), ((), ())),
            preferred_element_type=jnp.float32)
        logits = logits + b_ref[...]
        mx = jnp.max(logits, axis=0, keepdims=True)
        e = jnp.exp(logits - mx)
        p = e / jnp.sum(e, axis=0, keepdims=True)
        o_ref[...] = p[:LABELS, :]

    out = pl.pallas_call(
        body,
        grid=(B // BM,),
        in_specs=[
            pl.BlockSpec((BM, D), lambda i: (i, 0)),
            pl.BlockSpec((LP, D), lambda i: (0, 0)),
            pl.BlockSpec((LP, 1), lambda i: (0, 0)),
        ],
        out_specs=pl.BlockSpec((LABELS, BM), lambda i: (0, i)),
        out_shape=jax.ShapeDtypeStruct((LABELS, B), jnp.float32),
    )(m, Wp, bp)
    return out.T


def kernel(inputs, table, W, b):
    inputs = inputs.astype(jnp.int32)
    m = _sc_embed_mean(inputs, table)
    return _tc_head(m, W, b)
